# baseline (speedup 1.0000x reference)
import functools

import jax
import jax.numpy as jnp
from jax import lax
from jax.experimental import pallas as pl
from jax.experimental.pallas import tpu as pltpu
from jax.experimental.pallas import tpu_sc as plsc

KNN_K = 20
_NEG = -3.0e38


def _topk_body(R, N, KK, x_ref, xt_ref, idx_ref):
    xb = x_ref[0]
    xtr = xt_ref[0]
    x0 = xb[0:1, :]
    x1 = xb[1:2, :]
    x2 = xb[2:3, :]
    a0 = xtr[:, 0:1]
    a1 = xtr[:, 1:2]
    a2 = xtr[:, 2:3]
    xxc = (x0 * x0 + x1 * x1) + x2 * x2
    xxr = (a0 * a0 + a1 * a1) + a2 * a2
    a0b = a0.astype(jnp.bfloat16).astype(jnp.float32)
    a1b = a1.astype(jnp.bfloat16).astype(jnp.float32)
    a2b = a2.astype(jnp.bfloat16).astype(jnp.float32)
    x0b = x0.astype(jnp.bfloat16).astype(jnp.float32)
    x1b = x1.astype(jnp.bfloat16).astype(jnp.float32)
    x2b = x2.astype(jnp.bfloat16).astype(jnp.float32)
    g = (a0b * x0b + a1b * x1b) + a2b * x2b
    inner = -2.0 * g
    p = (-xxc - inner) - xxr

    Q = N // 4
    lane = lax.broadcasted_iota(jnp.int32, (R, 128), 1)
    iq = lax.broadcasted_iota(jnp.int32, (R, Q), 1)
    vals = [p[:, i * Q:(i + 1) * Q] for i in range(4)]
    idxs = [iq + i * Q for i in range(4)]

    def ce(i, j):
        vi, vj = vals[i], vals[j]
        xi, xj = idxs[i], idxs[j]
        swap = (vj > vi) | ((vj == vi) & (xj < xi))
        vals[i] = jnp.where(swap, vj, vi)
        vals[j] = jnp.where(swap, vi, vj)
        idxs[i] = jnp.where(swap, xj, xi)
        idxs[j] = jnp.where(swap, xi, xj)

    for a, b in ((0, 1), (2, 3), (0, 2), (1, 3), (1, 2)):
        ce(a, b)
    q0, q1, q2, q3 = vals
    p0, p1, p2, p3 = idxs
    bigi = jnp.int32(N)

    def body(k, carry):
        q0, q1, q2, q3, p0, p1, p2, jacc = carry
        m = jnp.max(q0, axis=1, keepdims=True)
        cand = jnp.where(q0 == m, p0, bigi)
        j = jnp.min(cand, axis=1, keepdims=True)
        hit = cand == j
        q0 = jnp.where(hit, q1, q0)
        q1 = jnp.where(hit, q2, q1)
        q2 = jnp.where(hit, q3, q2)
        q3 = jnp.where(hit, _NEG, q3)
        p0 = jnp.where(hit, p1, p0)
        p1 = jnp.where(hit, p2, p1)
        p2 = jnp.where(hit, p3, p2)
        jacc = jnp.where(lane == k, j, jacc)
        return q0, q1, q2, q3, p0, p1, p2, jacc

    jz = jnp.zeros((R, 128), jnp.int32)
    carry = (q0, q1, q2, q3, p0, p1, p2, jz)
    *_, jacc = lax.fori_loop(0, KK, body, carry)
    idx_ref[0] = jacc[:, :KK]


def _knn_topk(x, R=256, KK=KNN_K):
    B, C, N = x.shape
    xt = jnp.transpose(x, (0, 2, 1))
    T = N // R
    body = functools.partial(_topk_body, R, N, KK)
    return pl.pallas_call(
        body,
        grid=(B, T),
        in_specs=[
            pl.BlockSpec((1, C, N), lambda b, t: (b, 0, 0)),
            pl.BlockSpec((1, R, C), lambda b, t: (b, t, 0)),
        ],
        out_specs=pl.BlockSpec((1, R, KK), lambda b, t: (b, t, 0)),
        out_shape=jax.ShapeDtypeStruct((B, N, KK), jnp.int32),
    )(x, xt)


def _sc_gather(x, idx_flat, KK=KNN_K):
    B, C, N = x.shape
    info = plsc.get_sparse_core_info()
    NW = info.num_cores * info.num_subcores
    NC = info.num_cores
    CHUNKS = NW // B
    rows_per_w = N // CHUNKS
    PIECE = 128
    npieces = rows_per_w // PIECE
    mesh = plsc.VectorSubcoreMesh(core_axis_name="c", subcore_axis_name="s")

    @functools.partial(
        pl.kernel,
        mesh=mesh,
        compiler_params=pltpu.CompilerParams(needs_layout_passes=False),
        out_type=jax.ShapeDtypeStruct((B * 2 * C * N * KK,), jnp.float32),
        scratch_types=[
            pltpu.VMEM((N,), jnp.float32),
            pltpu.VMEM((N,), jnp.float32),
            pltpu.VMEM((N,), jnp.float32),
            pltpu.VMEM((PIECE * KK,), jnp.int32),
            pltpu.VMEM((2 * C, PIECE * KK), jnp.float32),
        ],
    )
    def k(x_hbm, idx_hbm, out_hbm, x0v, x1v, x2v, iv, ov):
        wid = lax.axis_index("s") * NC + lax.axis_index("c")
        b = wid // CHUNKS
        chunk = wid % CHUNKS
        pltpu.sync_copy(x_hbm.at[pl.ds((b * C + 0) * N, N)], x0v)
        pltpu.sync_copy(x_hbm.at[pl.ds((b * C + 1) * N, N)], x1v)
        pltpu.sync_copy(x_hbm.at[pl.ds((b * C + 2) * N, N)], x2v)
        iota = lax.iota(jnp.int32, 16)

        def piece_body(pc, carry):
            base_row = chunk * rows_per_w + pc * PIECE
            pltpu.sync_copy(
                idx_hbm.at[pl.ds((b * N + base_row) * KK, PIECE * KK)], iv)

            def group_body(g, carry2):
                for ch in range(5):
                    o = g * (4 * KK) + ch * 16
                    idxv = iv[pl.ds(o, 16)]
                    npat = (iota + ch * 16) // KK
                    nvec = (base_row + g * 4) + npat
                    for c, xv in ((0, x0v), (1, x1v), (2, x2v)):
                        cent = plsc.load_gather(xv, [nvec])
                        val = plsc.load_gather(xv, [idxv])
                        ov[c, pl.ds(o, 16)] = val - cent
                        ov[3 + c, pl.ds(o, 16)] = cent
                return carry2

            lax.fori_loop(0, PIECE // 4, group_body, 0)
            for c in range(2 * C):
                pltpu.sync_copy(
                    ov.at[c],
                    out_hbm.at[pl.ds(
                        ((b * 2 * C + c) * N + base_row) * KK, PIECE * KK)])
            return carry

        lax.fori_loop(0, npieces, piece_body, 0)

    return k(x.reshape(B * C * N), idx_flat.reshape(B * N * KK))


def kernel(x, mask):
    del mask
    B, C, N = x.shape
    idx = _knn_topk(x)
    feat = _sc_gather(x, idx)
    return feat.reshape(B, 2 * C, N, KNN_K)


# speedup vs baseline: 1.6770x; 1.6770x over previous
import functools

import jax
import jax.numpy as jnp
from jax import lax
from jax.experimental import pallas as pl
from jax.experimental.pallas import tpu as pltpu
from jax.experimental.pallas import tpu_sc as plsc

KNN_K = 20
_NEG = -3.0e38


def _topk_body(R, N, KK, x_ref, xt_ref, idx_ref):
    xb = x_ref[0]
    xtr = xt_ref[0]
    x0 = xb[0:1, :]
    x1 = xb[1:2, :]
    x2 = xb[2:3, :]
    a0 = xtr[:, 0:1]
    a1 = xtr[:, 1:2]
    a2 = xtr[:, 2:3]
    xxc = (x0 * x0 + x1 * x1) + x2 * x2
    xxr = (a0 * a0 + a1 * a1) + a2 * a2
    a0b = a0.astype(jnp.bfloat16).astype(jnp.float32)
    a1b = a1.astype(jnp.bfloat16).astype(jnp.float32)
    a2b = a2.astype(jnp.bfloat16).astype(jnp.float32)
    x0b = x0.astype(jnp.bfloat16).astype(jnp.float32)
    x1b = x1.astype(jnp.bfloat16).astype(jnp.float32)
    x2b = x2.astype(jnp.bfloat16).astype(jnp.float32)
    g = (a0b * x0b + a1b * x1b) + a2b * x2b
    inner = -2.0 * g
    p = (-xxc - inner) - xxr

    Q = N // 4
    lane = lax.broadcasted_iota(jnp.int32, (R, 128), 1)
    iq = lax.broadcasted_iota(jnp.int32, (R, Q), 1)
    vals = [p[:, i * Q:(i + 1) * Q] for i in range(4)]
    idxs = [iq + i * Q for i in range(4)]

    def ce(i, j):
        vi, vj = vals[i], vals[j]
        xi, xj = idxs[i], idxs[j]
        swap = (vj > vi) | ((vj == vi) & (xj < xi))
        vals[i] = jnp.where(swap, vj, vi)
        vals[j] = jnp.where(swap, vi, vj)
        idxs[i] = jnp.where(swap, xj, xi)
        idxs[j] = jnp.where(swap, xi, xj)

    for a, b in ((0, 1), (2, 3), (0, 2), (1, 3), (1, 2)):
        ce(a, b)
    q0, q1, q2, q3 = vals
    p0, p1, p2, p3 = idxs
    bigi = jnp.int32(N)

    UNROLL = 4

    def body(s, carry):
        q0, q1, q2, q3, p0, p1, p2, jacc = carry
        for u in range(UNROLL):
            m = jnp.max(q0, axis=1, keepdims=True)
            cand = jnp.where(q0 == m, p0, bigi)
            j = jnp.min(cand, axis=1, keepdims=True)
            hit = cand == j
            q0 = jnp.where(hit, q1, q0)
            q1 = jnp.where(hit, q2, q1)
            q2 = jnp.where(hit, q3, q2)
            q3 = jnp.where(hit, _NEG, q3)
            p0 = jnp.where(hit, p1, p0)
            p1 = jnp.where(hit, p2, p1)
            p2 = jnp.where(hit, p3, p2)
            jacc = jnp.where(lane == s * UNROLL + u, j, jacc)
        return q0, q1, q2, q3, p0, p1, p2, jacc

    jz = jnp.zeros((R, 128), jnp.int32)
    carry = (q0, q1, q2, q3, p0, p1, p2, jz)
    *_, jacc = lax.fori_loop(0, KK // UNROLL, body, carry)
    idx_ref[0] = jacc[:, :KK]


def _knn_topk(x, R=512, KK=KNN_K):
    B, C, N = x.shape
    xt = jnp.transpose(x, (0, 2, 1))
    T = N // R
    body = functools.partial(_topk_body, R, N, KK)
    return pl.pallas_call(
        body,
        grid=(B, T),
        in_specs=[
            pl.BlockSpec((1, C, N), lambda b, t: (b, 0, 0)),
            pl.BlockSpec((1, R, C), lambda b, t: (b, t, 0)),
        ],
        out_specs=pl.BlockSpec((1, R, KK), lambda b, t: (b, t, 0)),
        out_shape=jax.ShapeDtypeStruct((B, N, KK), jnp.int32),
    )(x, xt)


def _sc_gather(x, idx_flat, KK=KNN_K):
    B, C, N = x.shape
    info = plsc.get_sparse_core_info()
    NW = info.num_cores * info.num_subcores
    NC = info.num_cores
    CHUNKS = NW // B
    rows_per_w = N // CHUNKS
    PIECE = 128
    npieces = rows_per_w // PIECE
    mesh = plsc.VectorSubcoreMesh(core_axis_name="c", subcore_axis_name="s")

    @functools.partial(
        pl.kernel,
        mesh=mesh,
        compiler_params=pltpu.CompilerParams(needs_layout_passes=False),
        out_type=jax.ShapeDtypeStruct((B * 2 * C * N * KK,), jnp.float32),
        scratch_types=[
            pltpu.VMEM((N,), jnp.float32),
            pltpu.VMEM((N,), jnp.float32),
            pltpu.VMEM((N,), jnp.float32),
            pltpu.VMEM((PIECE * KK,), jnp.int32),
            pltpu.VMEM((2 * C, PIECE * KK), jnp.float32),
        ],
    )
    def k(x_hbm, idx_hbm, out_hbm, x0v, x1v, x2v, iv, ov):
        wid = lax.axis_index("s") * NC + lax.axis_index("c")
        b = wid // CHUNKS
        chunk = wid % CHUNKS
        pltpu.sync_copy(x_hbm.at[pl.ds((b * C + 0) * N, N)], x0v)
        pltpu.sync_copy(x_hbm.at[pl.ds((b * C + 1) * N, N)], x1v)
        pltpu.sync_copy(x_hbm.at[pl.ds((b * C + 2) * N, N)], x2v)
        iota = lax.iota(jnp.int32, 16)

        def piece_body(pc, carry):
            base_row = chunk * rows_per_w + pc * PIECE
            pltpu.sync_copy(
                idx_hbm.at[pl.ds((b * N + base_row) * KK, PIECE * KK)], iv)

            def group_body(g, carry2):
                for ch in range(5):
                    o = g * (4 * KK) + ch * 16
                    idxv = iv[pl.ds(o, 16)]
                    npat = (iota + ch * 16) // KK
                    nvec = (base_row + g * 4) + npat
                    for c, xv in ((0, x0v), (1, x1v), (2, x2v)):
                        cent = plsc.load_gather(xv, [nvec])
                        val = plsc.load_gather(xv, [idxv])
                        ov[c, pl.ds(o, 16)] = val - cent
                        ov[3 + c, pl.ds(o, 16)] = cent
                return carry2

            lax.fori_loop(0, PIECE // 4, group_body, 0)
            for c in range(2 * C):
                pltpu.sync_copy(
                    ov.at[c],
                    out_hbm.at[pl.ds(
                        ((b * 2 * C + c) * N + base_row) * KK, PIECE * KK)])
            return carry

        lax.fori_loop(0, npieces, piece_body, 0)

    return k(x.reshape(B * C * N), idx_flat.reshape(B * N * KK))


def kernel(x, mask):
    del mask
    B, C, N = x.shape
    idx = _knn_topk(x)
    feat = _sc_gather(x, idx)
    return feat.reshape(B, 2 * C, N, KNN_K)


# speedup vs baseline: 1.7882x; 1.0663x over previous
import functools

import jax
import jax.numpy as jnp
from jax import lax
from jax.experimental import pallas as pl
from jax.experimental.pallas import tpu as pltpu
from jax.experimental.pallas import tpu_sc as plsc

KNN_K = 20
_NEG = -3.0e38


def _topk_body(R, N, KK, x_ref, xt_ref, idx_ref):
    xb = x_ref[0]
    xtr = xt_ref[0]
    x0 = xb[0:1, :]
    x1 = xb[1:2, :]
    x2 = xb[2:3, :]
    a0 = xtr[:, 0:1]
    a1 = xtr[:, 1:2]
    a2 = xtr[:, 2:3]
    xxc = (x0 * x0 + x1 * x1) + x2 * x2
    xxr = (a0 * a0 + a1 * a1) + a2 * a2
    a0b = a0.astype(jnp.bfloat16).astype(jnp.float32)
    a1b = a1.astype(jnp.bfloat16).astype(jnp.float32)
    a2b = a2.astype(jnp.bfloat16).astype(jnp.float32)
    x0b = x0.astype(jnp.bfloat16).astype(jnp.float32)
    x1b = x1.astype(jnp.bfloat16).astype(jnp.float32)
    x2b = x2.astype(jnp.bfloat16).astype(jnp.float32)
    g = (a0b * x0b + a1b * x1b) + a2b * x2b
    inner = -2.0 * g
    p = (-xxc - inner) - xxr

    Q = N // 4
    lane = lax.broadcasted_iota(jnp.int32, (R, 128), 1)
    iq = lax.broadcasted_iota(jnp.int32, (R, Q), 1)
    vals = [p[:, i * Q:(i + 1) * Q] for i in range(4)]
    idxs = [iq + i * Q for i in range(4)]

    def ce(i, j):
        vi, vj = vals[i], vals[j]
        xi, xj = idxs[i], idxs[j]
        swap = (vj > vi) | ((vj == vi) & (xj < xi))
        vals[i] = jnp.where(swap, vj, vi)
        vals[j] = jnp.where(swap, vi, vj)
        idxs[i] = jnp.where(swap, xj, xi)
        idxs[j] = jnp.where(swap, xi, xj)

    for a, b in ((0, 1), (2, 3), (0, 2), (1, 3), (1, 2)):
        ce(a, b)
    q0, q1, q2, q3 = vals
    p0, p1, p2, p3 = idxs
    bigi = jnp.int32(N)

    UNROLL = 10

    def body(s, carry):
        q0, q1, q2, q3, p0, p1, p2, jacc = carry
        for u in range(UNROLL):
            m = jnp.max(q0, axis=1, keepdims=True)
            cand = jnp.where(q0 == m, p0, bigi)
            j = jnp.min(cand, axis=1, keepdims=True)
            hit = cand == j
            q0 = jnp.where(hit, q1, q0)
            q1 = jnp.where(hit, q2, q1)
            q2 = jnp.where(hit, q3, q2)
            q3 = jnp.where(hit, _NEG, q3)
            p0 = jnp.where(hit, p1, p0)
            p1 = jnp.where(hit, p2, p1)
            p2 = jnp.where(hit, p3, p2)
            jacc = jnp.where(lane == s * UNROLL + u, j, jacc)
        return q0, q1, q2, q3, p0, p1, p2, jacc

    jz = jnp.zeros((R, 128), jnp.int32)
    carry = (q0, q1, q2, q3, p0, p1, p2, jz)
    *_, jacc = lax.fori_loop(0, KK // UNROLL, body, carry)
    idx_ref[0] = jacc[:, :KK]


def _knn_topk(x, R=512, KK=KNN_K):
    B, C, N = x.shape
    xt = jnp.transpose(x, (0, 2, 1))
    T = N // R
    body = functools.partial(_topk_body, R, N, KK)
    return pl.pallas_call(
        body,
        grid=(B, T),
        in_specs=[
            pl.BlockSpec((1, C, N), lambda b, t: (b, 0, 0)),
            pl.BlockSpec((1, R, C), lambda b, t: (b, t, 0)),
        ],
        out_specs=pl.BlockSpec((1, R, KK), lambda b, t: (b, t, 0)),
        out_shape=jax.ShapeDtypeStruct((B, N, KK), jnp.int32),
    )(x, xt)


def _sc_gather(x, idx_flat, KK=KNN_K):
    B, C, N = x.shape
    info = plsc.get_sparse_core_info()
    NW = info.num_cores * info.num_subcores
    NC = info.num_cores
    CHUNKS = NW // B
    rows_per_w = N // CHUNKS
    PIECE = 128
    npieces = rows_per_w // PIECE
    mesh = plsc.VectorSubcoreMesh(core_axis_name="c", subcore_axis_name="s")

    @functools.partial(
        pl.kernel,
        mesh=mesh,
        compiler_params=pltpu.CompilerParams(needs_layout_passes=False),
        out_type=jax.ShapeDtypeStruct((B * 2 * C * N * KK,), jnp.float32),
        scratch_types=[
            pltpu.VMEM((N,), jnp.float32),
            pltpu.VMEM((N,), jnp.float32),
            pltpu.VMEM((N,), jnp.float32),
            pltpu.VMEM((PIECE * KK,), jnp.int32),
            pltpu.VMEM((2 * C, PIECE * KK), jnp.float32),
        ],
    )
    def k(x_hbm, idx_hbm, out_hbm, x0v, x1v, x2v, iv, ov):
        wid = lax.axis_index("s") * NC + lax.axis_index("c")
        b = wid // CHUNKS
        chunk = wid % CHUNKS
        pltpu.sync_copy(x_hbm.at[pl.ds((b * C + 0) * N, N)], x0v)
        pltpu.sync_copy(x_hbm.at[pl.ds((b * C + 1) * N, N)], x1v)
        pltpu.sync_copy(x_hbm.at[pl.ds((b * C + 2) * N, N)], x2v)
        iota = lax.iota(jnp.int32, 16)

        def piece_body(pc, carry):
            base_row = chunk * rows_per_w + pc * PIECE
            pltpu.sync_copy(
                idx_hbm.at[pl.ds((b * N + base_row) * KK, PIECE * KK)], iv)

            def group_body(g, carry2):
                for ch in range(5):
                    o = g * (4 * KK) + ch * 16
                    idxv = iv[pl.ds(o, 16)]
                    npat = (iota + ch * 16) // KK
                    nvec = (base_row + g * 4) + npat
                    for c, xv in ((0, x0v), (1, x1v), (2, x2v)):
                        cent = plsc.load_gather(xv, [nvec])
                        val = plsc.load_gather(xv, [idxv])
                        ov[c, pl.ds(o, 16)] = val - cent
                        ov[3 + c, pl.ds(o, 16)] = cent
                return carry2

            lax.fori_loop(0, PIECE // 4, group_body, 0)
            for c in range(2 * C):
                pltpu.sync_copy(
                    ov.at[c],
                    out_hbm.at[pl.ds(
                        ((b * 2 * C + c) * N + base_row) * KK, PIECE * KK)])
            return carry

        lax.fori_loop(0, npieces, piece_body, 0)

    return k(x.reshape(B * C * N), idx_flat.reshape(B * N * KK))


def kernel(x, mask):
    del mask
    B, C, N = x.shape
    idx = _knn_topk(x)
    feat = _sc_gather(x, idx)
    return feat.reshape(B, 2 * C, N, KNN_K)
